# Initial kernel scaffold; baseline (speedup 1.0000x reference)
import jax
import jax.numpy as jnp
from jax.experimental import pallas as pl


def kernel(input_labels, pos_labels, neg_labels, in_emb, out_emb):
    raise NotImplementedError("write your pallas kernel here")


# speedup vs baseline: 4.1037x; 4.1037x over previous
import functools

import jax
import jax.numpy as jnp
from jax import lax
from jax.experimental import pallas as pl
from jax.experimental.pallas import tpu as pltpu
from jax.experimental.pallas import tpu_sc as plsc

HIDDEN = 64
CTX = 128
GROUP = 4
NUM_WORKERS = 32


def _sc_dots_kernel(ex_per_w, u_labels_hbm, ctx_hbm, in_emb_hbm, out_emb_hbm,
                    out_hbm, u_idx, u_rows, ctx_idx, ctx_rows, dots, sem,
                    osem):
    wid = lax.axis_index("s") * 2 + lax.axis_index("c")
    base = wid * ex_per_w

    n_chunks = ex_per_w // 128
    for j in range(n_chunks):
        pltpu.sync_copy(u_labels_hbm.at[pl.ds(base + j * 128, 128)],
                        u_idx.at[j])
    cps = [pltpu.async_copy(in_emb_hbm.at[u_idx.at[j]],
                            u_rows.at[pl.ds(j * 128, 128)], sem)
           for j in range(n_chunks)]
    for cp in cps:
        cp.wait()

    def group_body(g, _):
        gb = base + g * GROUP
        pltpu.sync_copy(ctx_hbm.at[pl.ds(gb, GROUP)], ctx_idx)
        gcps = [pltpu.async_copy(out_emb_hbm.at[ctx_idx.at[e]],
                                 ctx_rows.at[e], sem)
                for e in range(GROUP)]
        for cp in gcps:
            cp.wait()
        lane = lax.iota(jnp.int32, 16)
        for e in range(GROUP):
            b_local = g * GROUP + e
            u_vecs = [u_rows[b_local, pl.ds(16 * h, 16)] for h in range(4)]

            def blk_body(t, _):
                acc = jnp.zeros((16,), jnp.float32)
                for r in range(16):
                    row = t * 16 + r
                    p = ctx_rows[e, row, pl.ds(0, 16)] * u_vecs[0]
                    for h in range(1, 4):
                        p = p + ctx_rows[e, row, pl.ds(16 * h, 16)] * u_vecs[h]
                    acc = jnp.where(lane == r, jnp.sum(p), acc)
                dots[e, pl.ds(t * 16, 16)] = acc
                return 0

            lax.fori_loop(0, CTX // 16, blk_body, 0)
        pltpu.async_copy(dots, out_hbm.at[pl.ds(gb, GROUP)], osem).wait()
        return 0

    lax.fori_loop(0, ex_per_w // GROUP, group_body, 0)


def _sc_dots(u_labels, ctx_labels, in_emb, out_emb):
    b = u_labels.shape[0]
    ex_per_w = b // NUM_WORKERS
    mesh = plsc.VectorSubcoreMesh(core_axis_name="c", subcore_axis_name="s")
    f = pl.kernel(
        functools.partial(_sc_dots_kernel, ex_per_w),
        out_type=jax.ShapeDtypeStruct((b, CTX), jnp.float32),
        mesh=mesh,
        scratch_types=[
            pltpu.VMEM((ex_per_w // 128, 128), jnp.int32),
            pltpu.VMEM((ex_per_w, HIDDEN), jnp.float32),
            pltpu.VMEM((GROUP, CTX), jnp.int32),
            pltpu.VMEM((GROUP, CTX, HIDDEN), jnp.float32),
            pltpu.VMEM((GROUP, CTX), jnp.float32),
            pltpu.SemaphoreType.DMA,
            pltpu.SemaphoreType.DMA,
        ],
        compiler_params=pltpu.CompilerParams(needs_layout_passes=False,
                                             use_tc_tiling_on_sc=False),
    )
    return f(u_labels, ctx_labels, in_emb, out_emb)


def _tc_loss_kernel(p, n, d_ref, o_ref):
    d = d_ref[...]
    col = lax.broadcasted_iota(jnp.int32, d.shape, 1)
    x = jnp.where(col < p, d, -d)
    ls = jnp.minimum(x, 0.0) - jnp.log1p(jnp.exp(-jnp.abs(x)))
    ls = jnp.where(col < p + n, ls, 0.0)
    loss = -jnp.sum(ls, axis=1)
    o_ref[...] = loss.reshape(o_ref.shape)


def _tc_loss(dots, p, n):
    b = dots.shape[0]
    blk = 2048
    out = pl.pallas_call(
        functools.partial(_tc_loss_kernel, p, n),
        grid=(b // blk,),
        in_specs=[pl.BlockSpec((blk, CTX), lambda i: (i, 0))],
        out_specs=pl.BlockSpec((blk // 128, 128), lambda i: (i, 0)),
        out_shape=jax.ShapeDtypeStruct((b // 128, 128), jnp.float32),
    )(dots)
    return out.reshape(b)


def kernel(input_labels, pos_labels, neg_labels, in_emb, out_emb):
    b, p = pos_labels.shape
    n = neg_labels.shape[1]
    pad = CTX - p - n
    ctx = jnp.concatenate(
        [pos_labels.astype(jnp.int32), neg_labels.astype(jnp.int32),
         jnp.zeros((b, pad), jnp.int32)], axis=1)
    dots = _sc_dots(input_labels.astype(jnp.int32), ctx, in_emb, out_emb)
    return _tc_loss(dots, p, n)
